# baseline (speedup 1.0000x reference)
import jax
import jax.numpy as jnp
from jax.experimental import pallas as pl

_ROWS_IN = 4096
_ROWS_OUT = 4095
_COLS = 8192
_CHUNK = 128
_NCHUNK = _COLS // _CHUNK
_BLK_R = 256


def _strict_upper(n, dtype):
    r = jax.lax.broadcasted_iota(jnp.int32, (n, n), 0)
    c = jax.lax.broadcasted_iota(jnp.int32, (n, n), 1)
    return (r < c).astype(dtype)


def _excl_cumsum_t_kernel(x_ref, o_ref):
    r = x_ref.shape[0]
    t128 = _strict_upper(_CHUNK, jnp.float32)
    t64 = _strict_upper(_NCHUNK, jnp.float32)

    x2 = x_ref[...].reshape(r * _NCHUNK, _CHUNK)
    excl_w = jnp.dot(x2, t128, preferred_element_type=jnp.float32)
    tots = jnp.sum(x2, axis=1).reshape(r, _NCHUNK)
    excl_t = jnp.dot(tots, t64, preferred_element_type=jnp.float32)

    out = excl_w.reshape(r, _NCHUNK, _CHUNK) + excl_t[:, :, None]
    o_ref[0:_COLS, :] = out.reshape(r, _COLS).T
    o_ref[_COLS:_COLS + 1, :] = (excl_t[:, _NCHUNK - 1]
                                 + tots[:, _NCHUNK - 1])[None, :]


@jax.jit
def kernel(x):
    grid = _ROWS_IN // _BLK_R
    t = pl.pallas_call(
        _excl_cumsum_t_kernel,
        grid=(grid,),
        in_specs=[pl.BlockSpec((_BLK_R, _COLS), lambda i: (i, 0))],
        out_specs=pl.BlockSpec((_COLS + 1, _BLK_R), lambda i: (0, i)),
        out_shape=jax.ShapeDtypeStruct((_COLS + 1, _ROWS_OUT), jnp.float32),
    )(x)
    return t.T
